# Initial kernel scaffold; baseline (speedup 1.0000x reference)
import jax
import jax.numpy as jnp
from jax.experimental import pallas as pl


def kernel(EA, e, q, r, inc_vects, inc_node_ids, inc_elem_ids):
    raise NotImplementedError("write your pallas kernel here")


# speedup vs baseline: 72.9486x; 72.9486x over previous
import jax
import jax.numpy as jnp
from jax import lax
from jax.experimental import pallas as pl
from jax.experimental.pallas import tpu as pltpu
from jax.experimental.pallas import tpu_sc as plsc

_B = 8
_N = 50000
_E = 800000
_I = 1600000

_NC, _NS, _L = 2, 16, 16
_NW = _NC * _NS
_PER_TILE = _I // _NW
_K = 80
_NSUB = _PER_TILE // _K
_STAGE = 125
_NSTG = _NSUB // _STAGE
_NPAD = 50176
_ZROWS = _NPAD // _NS
_ZCH = 392

_BLK_A = 6400
_BLK_F = 6250


def _table_body(ea_ref, ee_ref, t_ref):
    ax = ea_ref[...] * ee_ref[...]
    axt = ax.T
    t_ref[...] = jnp.concatenate([axt, axt], axis=1)


def _build_table(EA, e):
    return pl.pallas_call(
        _table_body,
        grid=(_E // _BLK_A,),
        in_specs=[pl.BlockSpec((_B, _BLK_A), lambda i: (0, i)),
                  pl.BlockSpec((_B, _BLK_A), lambda i: (0, i))],
        out_specs=pl.BlockSpec((_BLK_A, 2 * _B), lambda i: (i, 0)),
        out_shape=jax.ShapeDtypeStruct((_E, 2 * _B), jnp.float32),
    )(EA, e)


def _sc_body(t_hbm, eids_hbm, nids_hbm, vect_hbm, out_hbm,
             eidx_v, nidx_v, vect_v, rows_v, contrib_v, zbuf, acc, gsem):
    c = lax.axis_index("c")
    s = lax.axis_index("s")
    w = c * _NS + s

    lane = lax.iota(jnp.int32, _L)
    zero16 = jnp.zeros((_L,), jnp.float32)

    def zfill(i, carry):
        zbuf[i, :] = zero16
        return carry
    lax.fori_loop(0, _ZCH, zfill, 0)
    for zi in range(_ZROWS // _ZCH):
        pltpu.sync_copy(zbuf, acc.at[pl.ds(s * _ZROWS + zi * _ZCH, _ZCH)])
    plsc.subcore_barrier()

    pats = [jnp.where(lane < 8, 2 * t, 2 * t + 1) for t in range(8)]

    for b in range(_NSTG):
        pltpu.sync_copy(eids_hbm.at[w, b], eidx_v)
        pltpu.sync_copy(nids_hbm.at[w, b], nidx_v)
        pltpu.sync_copy(vect_hbm.at[w, b], vect_v)

        def jbody(j, carry):
            pltpu.async_copy(t_hbm.at[eidx_v.at[j]], rows_v, gsem).wait()
            for g in range(_K // 8):
                vv = vect_v[j, pl.ds(g * _L, _L)]
                for t in range(8):
                    i = g * 8 + t
                    mult = vv.at[pats[t]].get(mode="promise_in_bounds")
                    contrib_v[i, :] = rows_v[i, :] * mult
            pltpu.sync_copy(contrib_v, acc.at[nidx_v.at[j]], add=True)
            return carry
        lax.fori_loop(0, _STAGE, jbody, 0)

    plsc.subcore_barrier()
    pltpu.sync_copy(acc.at[pl.ds(s * _ZROWS, _ZROWS)],
                    out_hbm.at[c, pl.ds(s * _ZROWS, _ZROWS)])


def _sc_scatter(t_tab, eids2, nids2, vects2):
    mesh = plsc.VectorSubcoreMesh(core_axis_name="c", subcore_axis_name="s")
    kern = pl.kernel(
        _sc_body,
        out_type=jax.ShapeDtypeStruct((_NC, _NPAD, _L), jnp.float32),
        mesh=mesh,
        scratch_types=[
            pltpu.VMEM((_STAGE, _K), jnp.int32),
            pltpu.VMEM((_STAGE, _K), jnp.int32),
            pltpu.VMEM((_STAGE, 2 * _K), jnp.float32),
            pltpu.VMEM((_K, _L), jnp.float32),
            pltpu.VMEM((_K, _L), jnp.float32),
            pltpu.VMEM((_ZCH, _L), jnp.float32),
            pltpu.VMEM_SHARED((_NPAD, _L), jnp.float32),
            pltpu.SemaphoreType.DMA,
        ],
        compiler_params=pltpu.CompilerParams(use_tc_tiling_on_sc=False),
    )
    return kern(t_tab, eids2, nids2, vects2)


def _fin_body(a0_ref, a1_ref, q_ref, r_ref, o_ref):
    x = a0_ref[...] + a1_ref[...] - q_ref[...] - r_ref[...]

    @pl.when(pl.program_id(0) == 0)
    def _():
        o_ref[0, 0] = 0.0
    o_ref[0, 0] += jnp.sum(x * x)


def _finalize(a0, a1, q2, r2):
    nrow = _N * _B * 2 // 128
    return pl.pallas_call(
        _fin_body,
        grid=(nrow // _BLK_F,),
        in_specs=[pl.BlockSpec((_BLK_F, 128), lambda i: (i, 0))] * 4,
        out_specs=pl.BlockSpec(memory_space=pltpu.SMEM),
        out_shape=jax.ShapeDtypeStruct((1, 1), jnp.float32),
    )(a0, a1, q2, r2)


def kernel(EA, e, q, r, inc_vects, inc_node_ids, inc_elem_ids):
    t_tab = _build_table(EA, e)
    eids2 = inc_elem_ids.astype(jnp.int32).reshape(_NW, _NSTG, _STAGE, _K)
    nids2 = inc_node_ids.astype(jnp.int32).reshape(_NW, _NSTG, _STAGE, _K)
    vects2 = inc_vects.reshape(_NW, _NSTG, _STAGE, 2 * _K)
    acc = _sc_scatter(t_tab, eids2, nids2, vects2)
    accf = acc.reshape(_NC, _NPAD * _L // 128, 128)
    nrow = _N * _B * 2 // 128
    q2 = jnp.transpose(q, (1, 2, 0)).reshape(nrow, 128)
    r2 = jnp.transpose(r, (1, 2, 0)).reshape(nrow, 128)
    total = _finalize(accf[0, :nrow], accf[1, :nrow], q2, r2)
    return total[0, 0] / (_B * _N * 2)
